# baseline (speedup 1.0000x reference)
import jax
import jax.numpy as jnp
from jax import lax
from jax.experimental import pallas as pl
from jax.experimental.pallas import tpu as pltpu

B = 128
T = 32
P = 2048
D = 64
NB = 4
K = 2
TEMP = 8.0
HALF = P / 2.0


def _ring_kernel(xT_ref, in_Wt_ref, in_b_ref, out_Wt_ref, out_b_ref,
                 proc_Wt_ref, proc_b_ref, destT_ref, jump_Wc_ref, jump_b_ref,
                 cs_ref, pb_ref, ptr0_ref,
                 y_ref,
                 SU, W5, BASE, PTR, HID, CTX):
    L = proc_Wt_ref.shape[0]
    PTR[...] = ptr0_ref[...]
    HID[...] = jnp.zeros_like(HID)
    SU[...] = jnp.zeros_like(SU)
    W5[...] = jnp.zeros_like(W5)
    BASE[...] = jnp.full_like(BASE, -10.0)

    offs5 = lax.broadcasted_iota(jnp.int32, (5, B), 0).astype(jnp.float32) - K
    iotaT = lax.broadcasted_iota(jnp.int32, (T, 1), 0).astype(jnp.float32)
    iotaP = lax.broadcasted_iota(jnp.int32, (P, B), 0)

    def step_t(t, _):
        xt = xT_ref[t]
        inp = jnp.dot(in_Wt_ref[...], xt,
                      preferred_element_type=jnp.float32) + in_b_ref[...]
        for bi in range(NB):
            ptr = PTR[bi][None, :]
            base_i = jnp.clip(jnp.floor(ptr).astype(jnp.int32), 0, P - 1)
            base_f = base_i.astype(jnp.float32)
            idx_f = jnp.mod(base_f + offs5, P)
            delta = jnp.remainder(idx_f - ptr + HALF, P) - HALF
            logits = -(delta * delta) / TEMP
            mx = jnp.max(logits, axis=0, keepdims=True)
            ex = jnp.exp(logits - mx)
            w = ex / jnp.sum(ex, axis=0, keepdims=True)

            CTX[...] = jnp.zeros_like(CTX)
            for gi in range(NB):
                t_eff = t + 1 if gi < bi else t

                def group_body(gi=gi, t_eff=t_eff):
                    t_eff_f = t_eff.astype(jnp.float32)
                    dd = jnp.remainder(base_f - BASE[gi] + HALF, P) - HALF
                    validf = jnp.where(iotaT < t_eff_f, 1.0, 0.0)
                    near = (dd >= -4.0) & (dd <= 4.0)
                    anyv = jnp.max(jnp.where(near, validf, 0.0))

                    def corr_body():
                        c = jnp.zeros((T, B), jnp.float32)
                        for jp in range(5):
                            g = jnp.zeros((T, B), jnp.float32)
                            for m in range(5):
                                g = g + jnp.where(dd == float(jp - m),
                                                  w[m][None, :], 0.0)
                            c = c + W5[gi, jp] * g
                        c = c * validf
                        CTX[...] += jnp.sum(c[:, None, :] * SU[gi], axis=0)

                    pl.when(anyv > 0.5)(corr_body)

                if gi < bi or isinstance(t_eff, int):
                    group_body()
                else:
                    pl.when(t_eff > 0)(group_body)

            comb = inp + cs_ref[bi] * CTX[...] + 0.1 * pb_ref[bi]
            su = jnp.tanh(comb + HID[bi])
            for l in range(L):
                su = jnp.tanh(jnp.dot(proc_Wt_ref[l], su,
                                      preferred_element_type=jnp.float32)
                              + proc_b_ref[l])
            HID[bi] = su
            SU[bi, pl.ds(t, 1)] = su[None]
            W5[bi, :, pl.ds(t, 1), :] = w[:, None, :]
            BASE[bi, pl.ds(t, 1)] = base_f
            if bi == 0:
                ACC = su
            else:
                ACC = ACC + su

            jl = jnp.sum(jump_Wc_ref[bi] * su, axis=0, keepdims=True) \
                + jump_b_ref[bi]
            jd = jnp.where(jax.nn.sigmoid(jl) > 0.5, 1.0, 0.0)
            walk = jnp.remainder(ptr + 1.0, P)
            onehot = iotaP == base_i
            destv = jnp.sum(jnp.where(onehot, destT_ref[:, bi:bi + 1], 0.0),
                            axis=0, keepdims=True)
            PTR[bi] = jnp.remainder(jd * destv + (1.0 - jd) * walk, P)[0]
        y_ref[pl.ds(t, 1)] = (jnp.dot(out_Wt_ref[...], ACC * (1.0 / NB),
                                      preferred_element_type=jnp.float32)
                              + out_b_ref[...])[None]
        return 0

    lax.fori_loop(0, T, step_t, 0)


@jax.jit
def kernel(x, in_W, in_b, out_W, out_b, proc_W, proc_b, dest, jump_W, jump_b,
           ctx, phase, ptr_init):
    xT = jnp.transpose(x, (1, 2, 0))
    in_Wt = jnp.transpose(in_W)
    out_Wt = jnp.transpose(out_W)
    proc_Wt = jnp.transpose(proc_W, (0, 2, 1))
    destT = jnp.transpose(dest)
    pb = jnp.concatenate(
        [phase, jnp.zeros((NB, D - phase.shape[1]), phase.dtype)], axis=1)
    yT = pl.pallas_call(
        _ring_kernel,
        out_shape=jax.ShapeDtypeStruct((T, 8, B), jnp.float32),
        scratch_shapes=[
            pltpu.VMEM((NB, T, D, B), jnp.float32),
            pltpu.VMEM((NB, 5, T, B), jnp.float32),
            pltpu.VMEM((NB, T, B), jnp.float32),
            pltpu.VMEM((NB, B), jnp.float32),
            pltpu.VMEM((NB, D, B), jnp.float32),
            pltpu.VMEM((D, B), jnp.float32),
        ],
    )(xT, in_Wt, in_b[:, None], out_Wt, out_b[:, None],
      proc_Wt, proc_b[:, :, None], destT, jump_W[:, :, None],
      jump_b[:, None, None], jax.nn.sigmoid(ctx)[:, None, None],
      pb[:, :, None], ptr_init)
    return jnp.transpose(yT, (2, 0, 1))


# speedup vs baseline: 2.2713x; 2.2713x over previous
import jax
import jax.numpy as jnp
from jax import lax
from jax.experimental import pallas as pl
from jax.experimental.pallas import tpu as pltpu

B = 128
T = 32
P = 2048
D = 64
NB = 4
K = 2
TEMP = 8.0
E = T * NB
CHUNK = 32
HALF = P / 2.0


def _ring_kernel(xT_ref, in_Wt_ref, in_b_ref, out_Wt_ref, out_b_ref,
                 proc_Wt_ref, proc_b_ref, destRT_ref, jump_Wc_ref, jump_b_ref,
                 cs_ref, pb_ref, ptr0_ref,
                 y_ref,
                 SU, W5, BASE, PTR, HID):
    L = proc_Wt_ref.shape[0]
    PTR[...] = ptr0_ref[...]
    HID[...] = jnp.zeros_like(HID)
    SU[...] = jnp.zeros_like(SU)
    W5[...] = jnp.zeros_like(W5)
    BASE[...] = jnp.zeros_like(BASE)

    offs5 = lax.broadcasted_iota(jnp.int32, (5, B), 0).astype(jnp.float32) - K
    iotaC = lax.broadcasted_iota(jnp.int32, (CHUNK, 1), 0).astype(jnp.float32)
    iota16 = lax.broadcasted_iota(jnp.int32, (16, B), 0)
    iota128 = lax.broadcasted_iota(jnp.int32, (128, B), 0)

    def make_step(k):
        def step_t(t, _):
            xt = xT_ref[t]
            inp = jnp.dot(in_Wt_ref[...], xt,
                          preferred_element_type=jnp.float32) + in_b_ref[...]
            for bi in range(NB):
                ptr = PTR[bi][None, :]
                base_i = jnp.clip(jnp.floor(ptr).astype(jnp.int32), 0, P - 1)
                base_f = base_i.astype(jnp.float32)
                idx_f = jnp.mod(base_f + offs5, P)
                delta = jnp.remainder(idx_f - ptr + HALF, P) - HALF
                logits = -(delta * delta) / TEMP
                mx = jnp.max(logits, axis=0, keepdims=True)
                ex = jnp.exp(logits - mx)
                w = ex / jnp.sum(ex, axis=0, keepdims=True)

                context = jnp.zeros((D, B), jnp.float32)
                for ci in range(k + 1):
                    sl = slice(CHUNK * ci, CHUNK * (ci + 1))
                    dd = jnp.remainder(base_f - BASE[sl] + HALF, P) - HALF
                    c = jnp.zeros((CHUNK, B), jnp.float32)
                    for jp in range(5):
                        g = jnp.zeros((CHUNK, B), jnp.float32)
                        for m in range(5):
                            g = g + jnp.where(dd == float(jp - m),
                                              w[m][None, :], 0.0)
                        c = c + W5[jp, sl] * g
                    if ci == k:
                        s_rel = (t * NB + bi - CHUNK * k).astype(jnp.float32)
                        c = c * jnp.where(iotaC < s_rel, 1.0, 0.0)
                    context = context + jnp.sum(c[:, None, :] * SU[sl],
                                                axis=0)

                comb = inp + cs_ref[bi] * context + 0.1 * pb_ref[bi]
                su = jnp.tanh(comb + HID[bi])
                for l in range(L):
                    su = jnp.tanh(jnp.dot(proc_Wt_ref[l], su,
                                          preferred_element_type=jnp.float32)
                                  + proc_b_ref[l])
                HID[bi] = su
                SU[pl.ds(t * NB + bi, 1)] = su[None]
                W5[:, pl.ds(t * NB + bi, 1), :] = w[:, None, :]
                BASE[pl.ds(t * NB + bi, 1)] = base_f
                if bi == 0:
                    ACC = su
                else:
                    ACC = ACC + su

                jl = jnp.sum(jump_Wc_ref[bi] * su, axis=0, keepdims=True) \
                    + jump_b_ref[bi]
                jd = jnp.where(jax.nn.sigmoid(jl) > 0.5, 1.0, 0.0)
                walk = jnp.remainder(ptr + 1.0, P)
                hi = lax.div(base_i, 128)
                lo = base_i - hi * 128
                Mhi = jnp.where(iota16 == hi, 1.0, 0.0)
                dvals = jnp.dot(destRT_ref[bi], Mhi,
                                preferred_element_type=jnp.float32)
                Mlo = jnp.where(iota128 == lo, 1.0, 0.0)
                destv = jnp.sum(Mlo * dvals, axis=0, keepdims=True)
                PTR[bi] = jnp.remainder(jd * destv + (1.0 - jd) * walk, P)[0]
            y_ref[pl.ds(t, 1)] = (jnp.dot(out_Wt_ref[...], ACC * (1.0 / NB),
                                          preferred_element_type=jnp.float32)
                                  + out_b_ref[...])[None]
            return 0
        return step_t

    for k in range(4):
        lax.fori_loop(8 * k, 8 * (k + 1), make_step(k), 0)


@jax.jit
def kernel(x, in_W, in_b, out_W, out_b, proc_W, proc_b, dest, jump_W, jump_b,
           ctx, phase, ptr_init):
    xT = jnp.transpose(x, (1, 2, 0))
    in_Wt = jnp.transpose(in_W)
    out_Wt = jnp.transpose(out_W)
    proc_Wt = jnp.transpose(proc_W, (0, 2, 1))
    destRT = jnp.transpose(dest.reshape(NB, 16, 128), (0, 2, 1))
    pb = jnp.concatenate(
        [phase, jnp.zeros((NB, D - phase.shape[1]), phase.dtype)], axis=1)
    yT = pl.pallas_call(
        _ring_kernel,
        out_shape=jax.ShapeDtypeStruct((T, 8, B), jnp.float32),
        scratch_shapes=[
            pltpu.VMEM((E, D, B), jnp.float32),
            pltpu.VMEM((5, E, B), jnp.float32),
            pltpu.VMEM((E, B), jnp.float32),
            pltpu.VMEM((NB, B), jnp.float32),
            pltpu.VMEM((NB, D, B), jnp.float32),
        ],
    )(xT, in_Wt, in_b[:, None], out_Wt, out_b[:, None],
      proc_Wt, proc_b[:, :, None], destRT, jump_W[:, :, None],
      jump_b[:, None, None], jax.nn.sigmoid(ctx)[:, None, None],
      pb[:, :, None], ptr_init)
    return jnp.transpose(yT, (2, 0, 1))
